# baseline (speedup 1.0000x reference)
import functools

import jax
import jax.numpy as jnp
from jax import lax
from jax.experimental import pallas as pl
from jax.experimental.pallas import tpu as pltpu
from jax.experimental.pallas import tpu_sc as plsc

_SZ = (104, 96)
_OFF = (0, 104)
_NBUF = 8
_LAG = 4


def _emb_call(ids, table, b, s, d, nw, num_cores):
    rows_per_w = b // nw
    n_chunks = rows_per_w * 2
    mesh = plsc.VectorSubcoreMesh(core_axis_name="c", subcore_axis_name="s")

    @functools.partial(
        pl.kernel,
        mesh=mesh,
        compiler_params=pltpu.CompilerParams(use_tc_tiling_on_sc=False),
        out_type=jax.ShapeDtypeStruct((b, s, d), jnp.float32),
        scratch_types=[
            pltpu.VMEM((rows_per_w, s), jnp.int32),
            pltpu.VMEM((_NBUF, _SZ[0], d), jnp.float32),
            pltpu.SemaphoreType.DMA,
            pltpu.SemaphoreType.DMA,
        ],
    )
    def emb(ids_hbm, table_hbm, out_hbm, idx_v, rows_v, gsem, osem):
        wid = lax.axis_index("s") * num_cores + lax.axis_index("c")
        row0 = wid * rows_per_w
        pltpu.sync_copy(ids_hbm.at[pl.ds(row0, rows_per_w)], idx_v)

        def slot(u, p):
            return rows_v.at[u] if p == 0 else rows_v.at[u, pl.ds(0, _SZ[1])]

        def fire(c, u, p):
            r = c // 2
            pltpu.async_copy(
                table_hbm.at[idx_v.at[r, pl.ds(_OFF[p], _SZ[p])]],
                slot(u, p),
                gsem,
            )

        def drain_gather(u, p):
            pltpu.make_async_copy(
                table_hbm.at[pl.ds(0, _SZ[p])], slot(u, p), gsem
            ).wait()

        def put(c, u, p):
            r = c // 2
            pltpu.async_copy(
                slot(u, p),
                out_hbm.at[row0 + r, pl.ds(_OFF[p], _SZ[p])],
                osem,
            )

        def wait_put_one(p):
            pltpu.make_async_copy(
                table_hbm.at[pl.ds(0, _SZ[p])], slot(0, p), osem
            ).wait()

        for c in range(_LAG):
            fire(c, c % _NBUF, c % 2)
        for c in range(_LAG, _NBUF):
            fire(c, c % _NBUF, c % 2)
            drain_gather((c - _LAG) % _NBUF, (c - _LAG) % 2)
            put(c - _LAG, (c - _LAG) % _NBUF, (c - _LAG) % 2)

        def body(i, carry):
            c0 = _NBUF + i * _NBUF
            for j in range(_NBUF):
                c = c0 + j
                wait_put_one(j % 2)
                fire(c, j, j % 2)
                drain_gather((j - _LAG) % _NBUF, (j - _LAG) % 2)
                put(c - _LAG, (j - _LAG) % _NBUF, (j - _LAG) % 2)
            return carry

        lax.fori_loop(0, (n_chunks - _NBUF) // _NBUF, body, 0)

        for c in range(n_chunks, n_chunks + _LAG):
            u = (c - _LAG) % _NBUF
            drain_gather(u, (c - _LAG) % 2)
            put(c - _LAG, u, (c - _LAG) % 2)
        for k in range(_NBUF):
            wait_put_one(k % 2)

    return emb(ids, table)


@jax.jit
def kernel(source_ids, table):
    b, s = source_ids.shape
    d = table.shape[1]
    info = plsc.get_sparse_core_info()
    nw = info.num_cores * info.num_subcores
    ids = source_ids.astype(jnp.int32)
    return _emb_call(ids, table, b, s, d, nw, info.num_cores)
